# Initial kernel scaffold; baseline (speedup 1.0000x reference)
import jax
import jax.numpy as jnp
from jax.experimental import pallas as pl


def kernel(x, table):
    raise NotImplementedError("write your pallas kernel here")


# speedup vs baseline: 5.0865x; 5.0865x over previous
import functools

import jax
import jax.numpy as jnp
from jax import lax
from jax.experimental import pallas as pl
from jax.experimental.pallas import tpu as pltpu
from jax.experimental.pallas import tpu_sc as plsc

L = 16
NC, NS = 2, 16
NW = NC * NS
ROWS, COLS, D = 16384, 200, 5
N = ROWS * COLS
PER_W = N // NW
CH = 6400
NCH = PER_W // CH
OUT_CH = D * CH


def _sc_lookup(x_hbm, tbl_hbm, out_hbm, x_v, out_v, tbl_v):
    wid = lax.axis_index("s") * NC + lax.axis_index("c")
    pltpu.sync_copy(tbl_hbm, tbl_v)
    iota = lax.iota(jnp.int32, L)
    t0 = [tbl_v[d] for d in range(D)]
    dt = [tbl_v[D + d] for d in range(D)]
    base_idx = [iota * D + d for d in range(D)]

    def chunk_body(c, carry):
        start = wid * PER_W + c * CH
        pltpu.sync_copy(x_hbm.at[pl.ds(start, CH)], x_v)

        def group_body(g, carry2):
            xf = x_v[pl.ds(g * L, L)].astype(jnp.float32)
            off = g * (D * L)
            for d in range(D):
                plsc.store_scatter(out_v, [base_idx[d] + off],
                                   t0[d] + xf * dt[d])
            return carry2

        lax.fori_loop(0, CH // L, group_body, 0, unroll=4)
        pltpu.sync_copy(out_v, out_hbm.at[pl.ds(D * start, OUT_CH)])
        return carry

    lax.fori_loop(0, NCH, chunk_body, 0)


def kernel(x, table):
    t0_dt = jnp.concatenate([table[0], table[1] - table[0]])
    tbl16 = jnp.broadcast_to(t0_dt[:, None], (2 * D, L)).astype(jnp.float32)
    x_flat = x.reshape(-1)
    mesh = plsc.VectorSubcoreMesh(core_axis_name="c", subcore_axis_name="s")
    out = pl.kernel(
        _sc_lookup,
        out_type=jax.ShapeDtypeStruct((N * D,), jnp.float32),
        mesh=mesh,
        compiler_params=pltpu.CompilerParams(needs_layout_passes=False),
        scratch_types=[
            pltpu.VMEM((CH,), jnp.int32),
            pltpu.VMEM((OUT_CH,), jnp.float32),
            pltpu.VMEM((2 * D, L), jnp.float32),
        ],
    )(x_flat, tbl16)
    return out.reshape(ROWS, COLS, D)


# speedup vs baseline: 455.3654x; 89.5250x over previous
import jax
import jax.numpy as jnp
from jax.experimental import pallas as pl
from jax.experimental.pallas import tpu as pltpu

ROWS, COLS, D = 16384, 200, 5
BI = 1024


def _body(xt_ref, tbl_ref, y_ref):
    m = xt_ref[...] == 0
    for d in range(D):
        y_ref[d] = jnp.where(m, tbl_ref[d], tbl_ref[D + d])


def kernel(x, table):
    xt = x.T
    tblflat = jnp.concatenate([table[0], table[1]])
    y = pl.pallas_call(
        _body,
        grid=(ROWS // BI,),
        in_specs=[
            pl.BlockSpec((COLS, BI), lambda i: (0, i)),
            pl.BlockSpec(memory_space=pltpu.SMEM),
        ],
        out_specs=pl.BlockSpec((D, COLS, BI), lambda i: (0, 0, i)),
        out_shape=jax.ShapeDtypeStruct((D, COLS, ROWS), jnp.float32),
    )(xt, tblflat)
    return y.transpose(2, 1, 0)
